# baseline (speedup 1.0000x reference)
import functools

import jax
import jax.numpy as jnp
from jax import lax
from jax.experimental import pallas as pl
from jax.experimental.pallas import tpu as pltpu
from jax.experimental.pallas import tpu_sc as plsc

F = 26
V = 100000
E = 32
EPS = 1e-5

try:
    _info = plsc.get_sparse_core_info()
    _NC, _NS = _info.num_cores, _info.num_subcores
except Exception:
    _NC, _NS = 2, 16
_NW = _NC * _NS
_CHUNK = 128


def _sc_gather_body(idx_hbm, emb_hbm, fm_hbm, out_emb, out_fm,
                    idx_v, rows_v, fm_v, sem_e, sem_f, *, per_w):
    wid = lax.axis_index("s") * _NC + lax.axis_index("c")
    base = wid * per_w
    nchunk = per_w // _CHUNK
    pltpu.sync_copy(idx_hbm.at[wid], idx_v)

    copies = []
    for c in range(nchunk):
        copies.append(pltpu.async_copy(
            emb_hbm.at[idx_v.at[c]],
            rows_v.at[pl.ds(c * _CHUNK, _CHUNK)], sem_e))
        copies.append(pltpu.async_copy(
            fm_hbm.at[idx_v.at[c]], fm_v.at[c], sem_f))
    for cp in copies:
        cp.wait()
    pltpu.sync_copy(rows_v, out_emb.at[pl.ds(base, per_w)])
    pltpu.sync_copy(fm_v, out_fm.at[wid])


def _sc_gather(idx2, flat_e, flat_fm, n_rows):
    per_w = n_rows // _NW
    nchunk = per_w // _CHUNK
    mesh = plsc.VectorSubcoreMesh(core_axis_name="c", subcore_axis_name="s")
    kern = pl.kernel(
        functools.partial(_sc_gather_body, per_w=per_w),
        mesh=mesh,
        out_type=[
            jax.ShapeDtypeStruct((n_rows, E), jnp.float32),
            jax.ShapeDtypeStruct((_NW, nchunk, _CHUNK), jnp.float32),
        ],
        scratch_types=[
            pltpu.VMEM((nchunk, _CHUNK), jnp.int32),
            pltpu.VMEM((per_w, E), jnp.float32),
            pltpu.VMEM((nchunk, _CHUNK), jnp.float32),
            pltpu.SemaphoreType.DMA,
            pltpu.SemaphoreType.DMA,
        ],
        compiler_params=pltpu.CompilerParams(use_tc_tiling_on_sc=False),
    )
    return kern(idx2, flat_e, flat_fm)


def _mlp_body(x_ref, fmv_ref, w1_ref, w2_ref, wf_ref, b1_ref, g1_ref,
              be1_ref, rm1_ref, rv1_ref, b2_ref, g2_ref, be2_ref, rm2_ref,
              rv2_ref, bf_ref, o_ref):
    x = x_ref[...]
    z1 = jnp.dot(x, w1_ref[...], preferred_element_type=jnp.float32) + b1_ref[...]
    a1 = g1_ref[...] * lax.rsqrt(rv1_ref[...] + EPS)
    h1 = jnp.maximum(z1, 0.0) * a1 + (be1_ref[...] - rm1_ref[...] * a1)
    z2 = jnp.dot(h1, w2_ref[...], preferred_element_type=jnp.float32) + b2_ref[...]
    a2 = g2_ref[...] * lax.rsqrt(rv2_ref[...] + EPS)
    h2 = jnp.maximum(z2, 0.0) * a2 + (be2_ref[...] - rm2_ref[...] * a2)
    o = jnp.sum(h2 * wf_ref[...], axis=1, keepdims=True)
    r = lax.broadcasted_iota(jnp.int32, (F * E, E), 0) % E
    c = lax.broadcasted_iota(jnp.int32, (F * E, E), 1)
    a_mat = (r == c).astype(jnp.float32)
    s = jnp.dot(x, a_mat, preferred_element_type=jnp.float32)
    ss = jnp.dot(x * x, a_mat, preferred_element_type=jnp.float32)
    fm2 = 0.5 * jnp.sum(s * s - ss, axis=1, keepdims=True)
    fm1 = jnp.sum(fmv_ref[...], axis=1, keepdims=True)
    logit = o + bf_ref[...] + fm1 + fm2
    o_ref[...] = 1.0 / (1.0 + jnp.exp(-logit))


def _tc_mlp(dnn, fmv, w1p, w2t, wf, b1, g1, be1, rm1, rv1, b2, g2, be2,
            rm2, rv2, bf):
    b = dnn.shape[0]
    h1 = w1p.shape[1]
    h2 = w2t.shape[1]
    bt = 512
    grid = (b // bt,)
    full = lambda shape: pl.BlockSpec(shape, lambda i: (0, 0))
    return pl.pallas_call(
        _mlp_body,
        grid=grid,
        in_specs=[
            pl.BlockSpec((bt, F * E), lambda i: (i, 0)),
            pl.BlockSpec((bt, F), lambda i: (i, 0)),
            full((F * E, h1)),
            full((h1, h2)),
            full((1, h2)),
            full((1, h1)), full((1, h1)), full((1, h1)), full((1, h1)),
            full((1, h1)),
            full((1, h2)), full((1, h2)), full((1, h2)), full((1, h2)),
            full((1, h2)),
            full((1, 1)),
        ],
        out_specs=pl.BlockSpec((bt, 1), lambda i: (i, 0)),
        out_shape=jax.ShapeDtypeStruct((b, 1), jnp.float32),
    )(dnn, fmv, w1p, w2t, wf, b1, g1, be1, rm1, rv1, b2, g2, be2, rm2,
      rv2, bf)


def kernel(x, emb_tables, fm_tables, W1, b1, g1, be1, rm1, rv1,
           W2, b2, g2, be2, rm2, rv2, Wf, bf):
    b = x.shape[0]
    n_rows = b * F
    xi = x.astype(jnp.int32)
    idx = xi + (jnp.arange(F, dtype=jnp.int32) * V)[None, :]
    idx2 = idx.reshape(_NW, n_rows // (_NW * _CHUNK), _CHUNK)
    flat_e = emb_tables.reshape(F * V, E)
    flat_fm = fm_tables.reshape(F * V)

    emb_rows, fm_rows = _sc_gather(idx2, flat_e, flat_fm, n_rows)
    dnn = emb_rows.reshape(b, F * E)
    fmv = fm_rows.reshape(b, F)

    h1 = W1.shape[0]
    w1p = jnp.transpose(W1.reshape(h1, E, F), (2, 1, 0)).reshape(F * E, h1)
    w2t = W2.T
    r2 = lambda v: v.reshape(1, -1)
    return _tc_mlp(dnn, fmv, w1p, w2t, r2(Wf), r2(b1), r2(g1), r2(be1),
                   r2(rm1), r2(rv1), r2(b2), r2(g2), r2(be2), r2(rm2),
                   r2(rv2), bf.reshape(1, 1))


# speedup vs baseline: 1.4449x; 1.4449x over previous
import functools

import jax
import jax.numpy as jnp
from jax import lax
from jax.experimental import pallas as pl
from jax.experimental.pallas import tpu as pltpu
from jax.experimental.pallas import tpu_sc as plsc

F = 26
V = 100000
E = 32
EPS = 1e-5

try:
    _info = plsc.get_sparse_core_info()
    _NC, _NS = _info.num_cores, _info.num_subcores
except Exception:
    _NC, _NS = 2, 16
_NW = _NC * _NS
_CHUNK = 128


def _sc_gather_body(idxr_hbm, idxe_hbm, emb_hbm, fm_hbm, out_emb, out_fm,
                    idxr_v, idxe_v, buf_v, fm_v, sem_e, sem_f,
                    *, nrchunk, nechunk, nhalf):
    wid = lax.axis_index("s") * _NC + lax.axis_index("c")
    half = nechunk // nhalf

    pltpu.sync_copy(idxr_hbm.at[wid], idxr_v)
    fm_copies = []
    for c in range(nrchunk):
        fm_copies.append(pltpu.async_copy(
            fm_hbm.at[idxr_v.at[c]], fm_v.at[c], sem_f))

    for h in range(nhalf):
        pltpu.sync_copy(idxe_hbm.at[wid].at[pl.ds(h * half, half)], idxe_v)

        def _fire(j, _):
            pltpu.async_copy(emb_hbm.at[idxe_v.at[j]], buf_v.at[j], sem_e)
            return 0

        lax.fori_loop(0, half, _fire, 0)

        def _drain(j, _):
            pltpu.make_async_copy(emb_hbm.at[pl.ds(0, _CHUNK)],
                                  buf_v.at[0], sem_e).wait()
            return 0

        lax.fori_loop(0, half, _drain, 0)
        pltpu.sync_copy(buf_v, out_emb.at[wid].at[pl.ds(h * half, half)])

    for cp in fm_copies:
        cp.wait()
    pltpu.sync_copy(fm_v, out_fm.at[wid])


def _sc_gather(idxr, idxe, flat_e, flat_fm):
    nrchunk = idxr.shape[1]
    nechunk = idxe.shape[1]
    nhalf = 2
    mesh = plsc.VectorSubcoreMesh(core_axis_name="c", subcore_axis_name="s")
    kern = pl.kernel(
        functools.partial(_sc_gather_body, nrchunk=nrchunk,
                          nechunk=nechunk, nhalf=nhalf),
        mesh=mesh,
        out_type=[
            jax.ShapeDtypeStruct((_NW, nechunk, _CHUNK), jnp.float32),
            jax.ShapeDtypeStruct((_NW, nrchunk, _CHUNK), jnp.float32),
        ],
        scratch_types=[
            pltpu.VMEM((nrchunk, _CHUNK), jnp.int32),
            pltpu.VMEM((nechunk // nhalf, _CHUNK), jnp.int32),
            pltpu.VMEM((nechunk // nhalf, _CHUNK), jnp.float32),
            pltpu.VMEM((nrchunk, _CHUNK), jnp.float32),
            pltpu.SemaphoreType.DMA,
            pltpu.SemaphoreType.DMA,
        ],
        compiler_params=pltpu.CompilerParams(use_tc_tiling_on_sc=False),
    )
    return kern(idxr, idxe, flat_e, flat_fm)


def _mlp_body(x_ref, fmv_ref, w1_ref, w2_ref, wf_ref, b1_ref, g1_ref,
              be1_ref, rm1_ref, rv1_ref, b2_ref, g2_ref, be2_ref, rm2_ref,
              rv2_ref, bf_ref, o_ref):
    x = x_ref[...]
    z1 = jnp.dot(x, w1_ref[...], preferred_element_type=jnp.float32) + b1_ref[...]
    a1 = g1_ref[...] * lax.rsqrt(rv1_ref[...] + EPS)
    h1 = jnp.maximum(z1, 0.0) * a1 + (be1_ref[...] - rm1_ref[...] * a1)
    z2 = jnp.dot(h1, w2_ref[...], preferred_element_type=jnp.float32) + b2_ref[...]
    a2 = g2_ref[...] * lax.rsqrt(rv2_ref[...] + EPS)
    h2 = jnp.maximum(z2, 0.0) * a2 + (be2_ref[...] - rm2_ref[...] * a2)
    o = jnp.sum(h2 * wf_ref[...], axis=1, keepdims=True)
    r = lax.broadcasted_iota(jnp.int32, (F * E, E), 0) % E
    c = lax.broadcasted_iota(jnp.int32, (F * E, E), 1)
    a_mat = (r == c).astype(jnp.float32)
    s = jnp.dot(x, a_mat, preferred_element_type=jnp.float32)
    ss = jnp.dot(x * x, a_mat, preferred_element_type=jnp.float32)
    fm2 = 0.5 * jnp.sum(s * s - ss, axis=1, keepdims=True)
    fm1 = jnp.sum(fmv_ref[...], axis=1, keepdims=True)
    logit = o + bf_ref[...] + fm1 + fm2
    o_ref[...] = 1.0 / (1.0 + jnp.exp(-logit))


def _tc_mlp(dnn, fmv, w1p, w2t, wf, b1, g1, be1, rm1, rv1, b2, g2, be2,
            rm2, rv2, bf):
    b = dnn.shape[0]
    h1 = w1p.shape[1]
    h2 = w2t.shape[1]
    bt = 512
    grid = (b // bt,)
    full = lambda shape: pl.BlockSpec(shape, lambda i: (0, 0))
    return pl.pallas_call(
        _mlp_body,
        grid=grid,
        in_specs=[
            pl.BlockSpec((bt, F * E), lambda i: (i, 0)),
            pl.BlockSpec((bt, F), lambda i: (i, 0)),
            full((F * E, h1)),
            full((h1, h2)),
            full((1, h2)),
            full((1, h1)), full((1, h1)), full((1, h1)), full((1, h1)),
            full((1, h1)),
            full((1, h2)), full((1, h2)), full((1, h2)), full((1, h2)),
            full((1, h2)),
            full((1, 1)),
        ],
        out_specs=pl.BlockSpec((bt, 1), lambda i: (i, 0)),
        out_shape=jax.ShapeDtypeStruct((b, 1), jnp.float32),
    )(dnn, fmv, w1p, w2t, wf, b1, g1, be1, rm1, rv1, b2, g2, be2, rm2,
      rv2, bf)


def kernel(x, emb_tables, fm_tables, W1, b1, g1, be1, rm1, rv1,
           W2, b2, g2, be2, rm2, rv2, Wf, bf):
    b = x.shape[0]
    n_rows = b * F
    xi = x.astype(jnp.int32)
    idx = xi + (jnp.arange(F, dtype=jnp.int32) * V)[None, :]
    idxr = idx.reshape(_NW, n_rows // (_NW * _CHUNK), _CHUNK)
    idxe = (xi[:, :, None]
            + ((jnp.arange(F, dtype=jnp.int32) * E)[:, None]
               + jnp.arange(E, dtype=jnp.int32)[None, :])[None] * V)
    idxe = idxe.reshape(_NW, n_rows * E // (_NW * _CHUNK), _CHUNK)
    flat_e = jnp.transpose(emb_tables, (0, 2, 1)).reshape(F * E * V)
    flat_fm = fm_tables.reshape(F * V)

    emb_rows, fm_rows = _sc_gather(idxr, idxe, flat_e, flat_fm)
    dnn = emb_rows.reshape(b, F * E)
    fmv = fm_rows.reshape(b, F)

    h1 = W1.shape[0]
    w1p = jnp.transpose(W1.reshape(h1, E, F), (2, 1, 0)).reshape(F * E, h1)
    w2t = W2.T
    r2 = lambda v: v.reshape(1, -1)
    return _tc_mlp(dnn, fmv, w1p, w2t, r2(Wf), r2(b1), r2(g1), r2(be1),
                   r2(rm1), r2(rv1), r2(b2), r2(g2), r2(be2), r2(rm2),
                   r2(rv2), bf.reshape(1, 1))
